# Initial kernel scaffold; baseline (speedup 1.0000x reference)
import jax
import jax.numpy as jnp
from jax.experimental import pallas as pl


def kernel(bf, bl, adj, diff, W1, b1, a1, W2, b2, a2):
    raise NotImplementedError("write your pallas kernel here")


# speedup vs baseline: 1.0847x; 1.0847x over previous
import jax
import jax.numpy as jnp
from jax.experimental import pallas as pl
from jax.experimental.pallas import tpu as pltpu

N = 4096
D = 128
BM = 512


def _fused_gcn_kernel(adj_ref, diff_ref, bf_ref, bl_ref, w1_ref, b1_ref,
                      a1_ref, w2_ref, b2_ref, a2_ref, o_ref):
    agg1 = jnp.dot(adj_ref[...], bf_ref[...], preferred_element_type=jnp.float32)
    t1 = jnp.dot(agg1, w1_ref[...], preferred_element_type=jnp.float32) + b1_ref[...]
    agg2 = jnp.dot(diff_ref[...], bl_ref[...], preferred_element_type=jnp.float32)
    t2 = jnp.dot(agg2, w2_ref[...], preferred_element_type=jnp.float32) + b2_ref[...]
    a1 = a1_ref[0, 0]
    a2 = a2_ref[0, 0]
    o_ref[...] = (jnp.where(t1 >= 0, t1, a1 * t1)
                  + jnp.where(t2 >= 0, t2, a2 * t2))


def kernel(bf, bl, adj, diff, W1, b1, a1, W2, b2, a2):
    adj2 = adj.reshape(N, N)
    diff2 = diff.reshape(N, N)
    bf2 = bf.reshape(N, D)
    bl2 = bl.reshape(N, D)
    b1r = b1.reshape(1, D)
    b2r = b2.reshape(1, D)
    a1r = a1.reshape(1, 1)
    a2r = a2.reshape(1, 1)

    grid = (N // BM,)
    row_blk = pl.BlockSpec((BM, N), lambda i: (i, 0))
    const_nd = pl.BlockSpec((N, D), lambda i: (0, 0))
    const_dd = pl.BlockSpec((D, D), lambda i: (0, 0))
    const_1d = pl.BlockSpec((1, D), lambda i: (0, 0))
    const_11 = pl.BlockSpec((1, 1), lambda i: (0, 0))

    out = pl.pallas_call(
        _fused_gcn_kernel,
        grid=grid,
        in_specs=[row_blk, row_blk, const_nd, const_nd, const_dd, const_1d,
                  const_11, const_dd, const_1d, const_11],
        out_specs=pl.BlockSpec((BM, D), lambda i: (i, 0)),
        out_shape=jax.ShapeDtypeStruct((N, D), jnp.float32),
        compiler_params=pltpu.CompilerParams(
            dimension_semantics=("arbitrary",),
        ),
    )(adj2, diff2, bf2, bl2, W1, b1r, a1r, W2, b2r, a2r)
    return out.reshape(1, N, D)
